# baseline (speedup 1.0000x reference)
import functools

import jax
import jax.numpy as jnp
from jax import lax
from jax.experimental import pallas as pl
from jax.experimental.pallas import tpu as pltpu
from jax.experimental.pallas import tpu_sc as plsc

NC = 2
NS = 16
NW = NC * NS
LANES = 16


@functools.lru_cache(maxsize=None)
def _make_pool_kernel(B, L, D):
    XL = 128
    GLP = 56
    RPW = B // NW
    NREG = D // LANES
    NBUF = 8
    CHUNK = 64
    NCH = RPW // CHUNK
    DP = 2 * D

    mesh = plsc.VectorSubcoreMesh(core_axis_name="c", subcore_axis_name="s")

    @functools.partial(
        pl.kernel,
        mesh=mesh,
        compiler_params=pltpu.CompilerParams(use_tc_tiling_on_sc=False),
        out_type=jax.ShapeDtypeStruct((B, DP), jnp.float32),
        scratch_types=[
            pltpu.VMEM((2, CHUNK, XL), jnp.int32),
            pltpu.VMEM((NBUF, GLP, D), jnp.float32),
            pltpu.VMEM((RPW, DP), jnp.float32),
        ] + [pltpu.SemaphoreType.DMA] * NBUF,
    )
    def pool(x_hbm, table_hbm, out_hbm, idx_v, rows_v, pooled_v, *sems):
        cid = lax.axis_index("c")
        sid = lax.axis_index("s")
        wid = sid * NC + cid
        base = wid * RPW

        def stage(c, cb):
            pltpu.sync_copy(x_hbm.at[pl.ds(base + c * CHUNK, CHUNK)],
                            idx_v.at[cb])

        def fire(g, b, cb):
            pltpu.async_copy(table_hbm.at[idx_v.at[cb, g, pl.ds(0, GLP)]],
                             rows_v.at[b], sems[b])

        def consume(row, g, b, cb):
            pltpu.make_async_copy(table_hbm.at[idx_v.at[cb, g, pl.ds(0, GLP)]],
                                  rows_v.at[b], sems[b]).wait()

            def body(j, accs):
                out = accs
                for u in range(2):
                    r = 2 * j + u
                    out = tuple(out[k] + rows_v[b, r, pl.ds(k * LANES, LANES)]
                                for k in range(NREG))
                return out
            accs = tuple(jnp.zeros((LANES,), jnp.float32)
                         for _ in range(NREG))
            accs = lax.fori_loop(0, L // 2, body, accs)
            zero = jnp.zeros((LANES,), jnp.float32)
            for k in range(NREG):
                pooled_v[row, pl.ds(k * LANES, LANES)] = accs[k]
            for k in range(NREG, DP // LANES):
                pooled_v[row, pl.ds(k * LANES, LANES)] = zero

        stage(0, 0)
        for b in range(NBUF):
            fire(b, b, 0)

        def chunk_body(c, carry):
            cb = lax.rem(c, 2)
            nxt = lax.rem(c + 1, 2)

            @pl.when(c + 1 < NCH)
            def _():
                stage(c + 1, nxt)

            def step(i, carry2):
                for b in range(NBUF):
                    g = i * NBUF + b
                    consume(c * CHUNK + g, g, b, cb)
                    gg = g + NBUF
                    @pl.when(gg < CHUNK)
                    def _():
                        fire(gg, b, cb)

                    @pl.when(jnp.logical_and(gg >= CHUNK, c + 1 < NCH))
                    def _():
                        fire(gg - CHUNK, b, nxt)
                return carry2

            lax.fori_loop(0, CHUNK // NBUF, step, 0)
            return carry

        lax.fori_loop(0, NCH, chunk_body, 0)

        pltpu.sync_copy(pooled_v, out_hbm.at[pl.ds(base, RPW)])

    return pool


@functools.lru_cache(maxsize=None)
def _make_mlp_kernel(B, D, H, O, BT):
    def body(p_ref, w1_ref, b1_ref, w2_ref, b2_ref, o_ref):
        h = jnp.dot(p_ref[...], w1_ref[...],
                    preferred_element_type=jnp.float32) + b1_ref[...]
        h = jnp.maximum(h, 0.0)
        o_ref[...] = jnp.dot(h, w2_ref[...],
                             preferred_element_type=jnp.float32) + b2_ref[...]

    return pl.pallas_call(
        body,
        grid=(B // BT,),
        in_specs=[
            pl.BlockSpec((BT, D), lambda i: (i, 0)),
            pl.BlockSpec((D, H), lambda i: (0, 0)),
            pl.BlockSpec((1, H), lambda i: (0, 0)),
            pl.BlockSpec((H, O), lambda i: (0, 0)),
            pl.BlockSpec((1, O), lambda i: (0, 0)),
        ],
        out_specs=pl.BlockSpec((BT, O), lambda i: (i, 0)),
        out_shape=jax.ShapeDtypeStruct((B, O), jnp.float32),
    )


def kernel(x, table, W1, b1, W2, b2):
    B, L = x.shape
    V, D = table.shape
    H = W1.shape[0]
    O = W2.shape[0]
    xp = jnp.pad(x, ((0, 0), (0, 128 - L)))
    sums = _make_pool_kernel(B, L, D)(xp, table)
    w1p = jnp.concatenate([W1.T / float(L), jnp.zeros((D, H), W1.dtype)], axis=0)
    mlp = _make_mlp_kernel(B, 2 * D, H, O, 2048)
    return mlp(sums, w1p, b1.reshape(1, H), W2.T, b2.reshape(1, O))


# speedup vs baseline: 3.5496x; 3.5496x over previous
import functools

import jax
import jax.numpy as jnp
from jax import lax
from jax.experimental import pallas as pl
from jax.experimental.pallas import tpu as pltpu
from jax.experimental.pallas import tpu_sc as plsc

NC = 2
NS = 16
NW = NC * NS
LANES = 16


@functools.lru_cache(maxsize=None)
def _make_pool_kernel(B, L, D):
    XL = 128
    GLP = 56
    RPW = B // NW
    NREG = D // LANES
    NBUF = 8
    CHUNK = 64
    NCH = RPW // CHUNK
    DP = 2 * D

    mesh = plsc.VectorSubcoreMesh(core_axis_name="c", subcore_axis_name="s")

    @functools.partial(
        pl.kernel,
        mesh=mesh,
        compiler_params=pltpu.CompilerParams(use_tc_tiling_on_sc=False),
        out_type=jax.ShapeDtypeStruct((B, DP), jnp.float32),
        scratch_types=[
            pltpu.VMEM((2, CHUNK, XL), jnp.int32),
            pltpu.VMEM((NBUF, GLP, D), jnp.float32),
            pltpu.VMEM((RPW, DP), jnp.float32),
        ] + [pltpu.SemaphoreType.DMA] * NBUF,
    )
    def pool(x_hbm, table_hbm, out_hbm, idx_v, rows_v, pooled_v, *sems):
        cid = lax.axis_index("c")
        sid = lax.axis_index("s")
        wid = sid * NC + cid
        base = wid * RPW

        def stage(c, cb):
            pltpu.sync_copy(x_hbm.at[pl.ds(base + c * CHUNK, CHUNK)],
                            idx_v.at[cb])

        def fire(g, b, cb):
            pltpu.async_copy(table_hbm.at[idx_v.at[cb, g, pl.ds(0, GLP)]],
                             rows_v.at[b], sems[b])

        def consume(row, g, b, cb):
            pltpu.make_async_copy(table_hbm.at[idx_v.at[cb, g, pl.ds(0, GLP)]],
                                  rows_v.at[b], sems[b]).wait()

            def body(j, accs):
                out = accs
                for u in range(2):
                    r = 2 * j + u
                    out = tuple(out[k] + rows_v[b, r, pl.ds(k * LANES, LANES)]
                                for k in range(NREG))
                return out
            accs = tuple(jnp.zeros((LANES,), jnp.float32)
                         for _ in range(NREG))
            accs = lax.fori_loop(0, L // 2, body, accs)
            zero = jnp.zeros((LANES,), jnp.float32)
            for k in range(NREG):
                pooled_v[row, pl.ds(k * LANES, LANES)] = accs[k]
            for k in range(NREG, DP // LANES):
                pooled_v[row, pl.ds(k * LANES, LANES)] = zero

        stage(0, 0)
        for b in range(NBUF):
            fire(b, b, 0)

        def chunk_body(c, carry):
            cb = lax.rem(c, 2)
            nxt = lax.rem(c + 1, 2)

            @pl.when(c + 1 < NCH)
            def _():
                stage(c + 1, nxt)

            def step(i, carry2):
                for b in range(NBUF):
                    g = i * NBUF + b
                    consume(c * CHUNK + g, g, b, cb)
                    gg = g + NBUF
                    @pl.when(gg < CHUNK)
                    def _():
                        fire(gg, b, cb)

                    @pl.when(jnp.logical_and(gg >= CHUNK, c + 1 < NCH))
                    def _():
                        fire(gg - CHUNK, b, nxt)
                return carry2

            lax.fori_loop(0, CHUNK // NBUF, step, 0)
            return carry

        lax.fori_loop(0, NCH, chunk_body, 0)

        pltpu.sync_copy(pooled_v, out_hbm.at[pl.ds(base, RPW)])

    return pool


@functools.lru_cache(maxsize=None)
def _make_mlp_kernel(B, D, H, O, BT):
    def body(p_ref, w1_ref, b1_ref, w2_ref, b2_ref, o_ref):
        h = jnp.dot(p_ref[...], w1_ref[...],
                    preferred_element_type=jnp.float32) + b1_ref[...]
        h = jnp.maximum(h, 0.0)
        o_ref[...] = jnp.dot(h, w2_ref[...],
                             preferred_element_type=jnp.float32) + b2_ref[...]

    return pl.pallas_call(
        body,
        grid=(B // BT,),
        in_specs=[
            pl.BlockSpec((BT, D), lambda i: (i, 0)),
            pl.BlockSpec((D, H), lambda i: (0, 0)),
            pl.BlockSpec((1, H), lambda i: (0, 0)),
            pl.BlockSpec((H, O), lambda i: (0, 0)),
            pl.BlockSpec((1, O), lambda i: (0, 0)),
        ],
        out_specs=pl.BlockSpec((BT, O), lambda i: (i, 0)),
        out_shape=jax.ShapeDtypeStruct((B, O), jnp.float32),
    )


def kernel(x, table, W1, b1, W2, b2):
    B, L = x.shape
    V, D = table.shape
    H = W1.shape[0]
    O = W2.shape[0]
    xp = jnp.concatenate(
        [x, x[:, :6], jnp.zeros((B, 72), x.dtype)], axis=1)
    sums = _make_pool_kernel(B, L, D)(xp, table)
    w1p = jnp.concatenate([W1.T / float(L), jnp.zeros((D, H), W1.dtype)], axis=0)
    mlp = _make_mlp_kernel(B, 2 * D, H, O, 2048)
    return mlp(sums, w1p, b1.reshape(1, H), W2.T, b2.reshape(1, O))
